# Initial kernel scaffold; baseline (speedup 1.0000x reference)
import jax
import jax.numpy as jnp
from jax.experimental import pallas as pl


def kernel(xyz1, xyz2):
    raise NotImplementedError("write your pallas kernel here")


# speedup vs baseline: 1.9648x; 1.9648x over previous
import jax
import jax.numpy as jnp
from jax.experimental import pallas as pl
from jax.experimental.pallas import tpu as pltpu

_TN = 512


def _chamfer_block(x1_ref, x2t_ref, d1_ref, i1_ref, d2_ref, i2_ref):
    i = pl.program_id(1)
    a = x1_ref[0]
    bt = x2t_ref[0]
    n = bt.shape[1]

    d = None
    for k in range(3):
        ak = a[:, k : k + 1]
        bk = bt[k : k + 1, :]
        t = ak - bk
        t = t * t
        d = t if d is None else d + t

    m1 = jnp.min(d, axis=1, keepdims=True)
    jiota = jax.lax.broadcasted_iota(jnp.int32, d.shape, 1)
    a1 = jnp.min(jnp.where(d == m1, jiota, n), axis=1, keepdims=True)
    d1_ref[0] = m1.T
    i1_ref[0] = a1.T

    m2 = jnp.min(d, axis=0, keepdims=True)
    riota = jax.lax.broadcasted_iota(jnp.int32, d.shape, 0) + i * _TN
    a2 = jnp.min(jnp.where(d == m2, riota, n), axis=0, keepdims=True)

    @pl.when(i == 0)
    def _():
        d2_ref[0] = m2
        i2_ref[0] = a2

    @pl.when(i > 0)
    def _():
        prev_d = d2_ref[0]
        prev_i = i2_ref[0]
        better = m2 < prev_d
        d2_ref[0] = jnp.where(better, m2, prev_d)
        i2_ref[0] = jnp.where(better, a2, prev_i)


def kernel(xyz1, xyz2):
    b, n, _ = xyz1.shape
    nb = n // _TN
    x2t = jnp.transpose(xyz2, (0, 2, 1))
    d1, i1, d2, i2 = pl.pallas_call(
        _chamfer_block,
        grid=(b, nb),
        in_specs=[
            pl.BlockSpec((1, _TN, 3), lambda bb, ii: (bb, ii, 0)),
            pl.BlockSpec((1, 3, n), lambda bb, ii: (bb, 0, 0)),
        ],
        out_specs=[
            pl.BlockSpec((1, 1, _TN), lambda bb, ii: (bb * nb + ii, 0, 0)),
            pl.BlockSpec((1, 1, _TN), lambda bb, ii: (bb * nb + ii, 0, 0)),
            pl.BlockSpec((1, 1, n), lambda bb, ii: (bb, 0, 0)),
            pl.BlockSpec((1, 1, n), lambda bb, ii: (bb, 0, 0)),
        ],
        out_shape=[
            jax.ShapeDtypeStruct((b * nb, 1, _TN), jnp.float32),
            jax.ShapeDtypeStruct((b * nb, 1, _TN), jnp.int32),
            jax.ShapeDtypeStruct((b, 1, n), jnp.float32),
            jax.ShapeDtypeStruct((b, 1, n), jnp.int32),
        ],
        compiler_params=pltpu.CompilerParams(
            dimension_semantics=("parallel", "arbitrary"),
        ),
    )(xyz1, x2t)
    return (
        d1.reshape(b, n),
        d2.reshape(b, n),
        i1.reshape(b, n),
        i2.reshape(b, n),
    )


# speedup vs baseline: 1.9842x; 1.0099x over previous
import jax
import jax.numpy as jnp
from jax.experimental import pallas as pl
from jax.experimental.pallas import tpu as pltpu

_TN = 1024


def _chamfer_block(x1_ref, x2t_ref, d1_ref, i1_ref, d2_ref, i2_ref):
    i = pl.program_id(1)
    a = x1_ref[0]
    bt = x2t_ref[0]
    n = bt.shape[1]

    d = None
    for k in range(3):
        ak = a[:, k : k + 1]
        bk = bt[k : k + 1, :]
        t = ak - bk
        t = t * t
        d = t if d is None else d + t

    m1 = jnp.min(d, axis=1, keepdims=True)
    jiota = jax.lax.broadcasted_iota(jnp.int32, d.shape, 1)
    a1 = jnp.min(jnp.where(d == m1, jiota, n), axis=1, keepdims=True)
    d1_ref[0] = m1.T
    i1_ref[0] = a1.T

    m2 = jnp.min(d, axis=0, keepdims=True)
    riota = jax.lax.broadcasted_iota(jnp.int32, d.shape, 0) + i * _TN
    a2 = jnp.min(jnp.where(d == m2, riota, n), axis=0, keepdims=True)

    @pl.when(i == 0)
    def _():
        d2_ref[0] = m2
        i2_ref[0] = a2

    @pl.when(i > 0)
    def _():
        prev_d = d2_ref[0]
        prev_i = i2_ref[0]
        better = m2 < prev_d
        d2_ref[0] = jnp.where(better, m2, prev_d)
        i2_ref[0] = jnp.where(better, a2, prev_i)


def kernel(xyz1, xyz2):
    b, n, _ = xyz1.shape
    nb = n // _TN
    x2t = jnp.transpose(xyz2, (0, 2, 1))
    d1, i1, d2, i2 = pl.pallas_call(
        _chamfer_block,
        grid=(b, nb),
        in_specs=[
            pl.BlockSpec((1, _TN, 3), lambda bb, ii: (bb, ii, 0)),
            pl.BlockSpec((1, 3, n), lambda bb, ii: (bb, 0, 0)),
        ],
        out_specs=[
            pl.BlockSpec((1, 1, _TN), lambda bb, ii: (bb * nb + ii, 0, 0)),
            pl.BlockSpec((1, 1, _TN), lambda bb, ii: (bb * nb + ii, 0, 0)),
            pl.BlockSpec((1, 1, n), lambda bb, ii: (bb, 0, 0)),
            pl.BlockSpec((1, 1, n), lambda bb, ii: (bb, 0, 0)),
        ],
        out_shape=[
            jax.ShapeDtypeStruct((b * nb, 1, _TN), jnp.float32),
            jax.ShapeDtypeStruct((b * nb, 1, _TN), jnp.int32),
            jax.ShapeDtypeStruct((b, 1, n), jnp.float32),
            jax.ShapeDtypeStruct((b, 1, n), jnp.int32),
        ],
        compiler_params=pltpu.CompilerParams(
            dimension_semantics=("parallel", "arbitrary"),
        ),
    )(xyz1, x2t)
    return (
        d1.reshape(b, n),
        d2.reshape(b, n),
        i1.reshape(b, n),
        i2.reshape(b, n),
    )


# speedup vs baseline: 1.9869x; 1.0014x over previous
import jax
import jax.numpy as jnp
from jax.experimental import pallas as pl
from jax.experimental.pallas import tpu as pltpu

_TN = 1024


def _chamfer_block(x1_ref, x2t_ref, d1_ref, i1_ref, d2_ref, i2_ref):
    i = pl.program_id(1)
    a = x1_ref[0]
    bt = x2t_ref[0]
    n = bt.shape[1]

    d = None
    for k in range(3):
        ak = a[:, k : k + 1]
        bk = bt[k : k + 1, :]
        t = ak - bk
        t = t * t
        d = t if d is None else d + t

    m1 = jnp.min(d, axis=1, keepdims=True)
    jiota = jax.lax.broadcasted_iota(jnp.int32, d.shape, 1)
    a1 = jnp.min(jnp.where(d == m1, jiota, n), axis=1, keepdims=True)
    d1_ref[0] = m1.T
    i1_ref[0] = a1.T

    m2 = jnp.min(d, axis=0, keepdims=True)
    riota = jax.lax.broadcasted_iota(jnp.int32, d.shape, 0)
    a2 = jnp.min(jnp.where(d == m2, riota, n), axis=0, keepdims=True) + i * _TN

    @pl.when(i == 0)
    def _():
        d2_ref[0] = m2
        i2_ref[0] = a2

    @pl.when(i > 0)
    def _():
        prev_d = d2_ref[0]
        prev_i = i2_ref[0]
        better = m2 < prev_d
        d2_ref[0] = jnp.where(better, m2, prev_d)
        i2_ref[0] = jnp.where(better, a2, prev_i)


def kernel(xyz1, xyz2):
    b, n, _ = xyz1.shape
    nb = n // _TN
    x2t = jnp.transpose(xyz2, (0, 2, 1))
    d1, i1, d2, i2 = pl.pallas_call(
        _chamfer_block,
        grid=(b, nb),
        in_specs=[
            pl.BlockSpec((1, _TN, 3), lambda bb, ii: (bb, ii, 0)),
            pl.BlockSpec((1, 3, n), lambda bb, ii: (bb, 0, 0)),
        ],
        out_specs=[
            pl.BlockSpec((1, 1, _TN), lambda bb, ii: (bb * nb + ii, 0, 0)),
            pl.BlockSpec((1, 1, _TN), lambda bb, ii: (bb * nb + ii, 0, 0)),
            pl.BlockSpec((1, 1, n), lambda bb, ii: (bb, 0, 0)),
            pl.BlockSpec((1, 1, n), lambda bb, ii: (bb, 0, 0)),
        ],
        out_shape=[
            jax.ShapeDtypeStruct((b * nb, 1, _TN), jnp.float32),
            jax.ShapeDtypeStruct((b * nb, 1, _TN), jnp.int32),
            jax.ShapeDtypeStruct((b, 1, n), jnp.float32),
            jax.ShapeDtypeStruct((b, 1, n), jnp.int32),
        ],
        compiler_params=pltpu.CompilerParams(
            dimension_semantics=("parallel", "arbitrary"),
        ),
    )(xyz1, x2t)
    return (
        d1.reshape(b, n),
        d2.reshape(b, n),
        i1.reshape(b, n),
        i2.reshape(b, n),
    )
